# baseline (speedup 1.0000x reference)
import jax
import jax.numpy as jnp
from jax import lax
from jax.experimental import pallas as pl
from jax.experimental.pallas import tpu as pltpu
from jax.experimental.pallas import tpu_sc as plsc

SENT_LEN = 200
D = 64
NUM_WORKERS = 32
IDX_MINOR = 100
SENT_PER_CHUNK = 4
CHUNK = SENT_PER_CHUNK * SENT_LEN
LANES = 16


def _sc_body(x2, xflat, table, pos2, postab, out, mask_out,
             pos_idx, pos_v, idx_g, idx_m, rows_v, mask_v, sem):
    wid = lax.axis_index("s") * 2 + lax.axis_index("c")

    pltpu.sync_copy(pos2, pos_idx)
    for j in range(SENT_LEN // IDX_MINOR):
        pltpu.async_copy(postab.at[pos_idx.at[j]],
                         pos_v.at[pl.ds(j * IDX_MINOR, IDX_MINOR)],
                         sem).wait()

    tokens_per_worker = x2.shape[0] * IDX_MINOR // NUM_WORKERS
    n_steps = tokens_per_worker // CHUNK
    idx_rows_per_chunk = CHUNK // IDX_MINOR

    def step(i, carry):
        base_tok = pl.multiple_of(wid * tokens_per_worker + i * CHUNK, CHUNK)
        base_row = pl.multiple_of(base_tok // IDX_MINOR, CHUNK // IDX_MINOR)

        pltpu.sync_copy(x2.at[pl.ds(base_row, idx_rows_per_chunk)], idx_g)
        pltpu.sync_copy(xflat.at[pl.ds(base_tok, CHUNK)], idx_m)

        copies = [
            pltpu.async_copy(table.at[idx_g.at[j]],
                             rows_v.at[pl.ds(j * IDX_MINOR, IDX_MINOR)],
                             sem)
            for j in range(idx_rows_per_chunk)
        ]
        for c in copies:
            c.wait()

        def add_body(r, c2):
            for s in range(SENT_PER_CHUNK):
                for col in range(D // LANES):
                    sl = pl.ds(col * LANES, LANES)
                    rows_v[s * SENT_LEN + r, sl] = (
                        rows_v[s * SENT_LEN + r, sl] + pos_v[r, sl])
            return c2
        lax.fori_loop(0, SENT_LEN, add_body, 0)

        for j in range(CHUNK // LANES):
            sl = pl.ds(j * LANES, LANES)
            v = idx_m[sl]
            mask_v[sl] = jnp.where(v == jnp.int32(0), jnp.int32(1),
                                   jnp.int32(0))

        pltpu.sync_copy(rows_v, out.at[pl.ds(base_tok, CHUNK)])
        pltpu.sync_copy(mask_v, mask_out.at[pl.ds(base_tok, CHUNK)])
        return carry

    lax.fori_loop(0, n_steps, step, 0)


def kernel(x, token_table, pos_table, positions):
    B, L = x.shape
    n_tok = B * L
    x32 = x.astype(jnp.int32)
    x2 = x32.reshape(n_tok // IDX_MINOR, IDX_MINOR)
    xflat = x32.reshape(n_tok)
    pos2 = positions.astype(jnp.int32).reshape(L // IDX_MINOR, IDX_MINOR)

    mesh = plsc.VectorSubcoreMesh(core_axis_name="c", subcore_axis_name="s")
    out, mask_i32 = pl.kernel(
        _sc_body,
        out_type=[
            jax.ShapeDtypeStruct((n_tok, D), jnp.float32),
            jax.ShapeDtypeStruct((n_tok,), jnp.int32),
        ],
        mesh=mesh,
        compiler_params=pltpu.CompilerParams(use_tc_tiling_on_sc=False),
        scratch_types=[
            pltpu.VMEM((L // IDX_MINOR, IDX_MINOR), jnp.int32),
            pltpu.VMEM((SENT_LEN, D), jnp.float32),
            pltpu.VMEM((CHUNK // IDX_MINOR, IDX_MINOR), jnp.int32),
            pltpu.VMEM((CHUNK,), jnp.int32),
            pltpu.VMEM((CHUNK, D), jnp.float32),
            pltpu.VMEM((CHUNK,), jnp.int32),
            pltpu.SemaphoreType.DMA,
        ],
    )(x2, xflat, token_table, pos2, pos_table)

    return (out.reshape(B, L, D), mask_i32.astype(jnp.bool_).reshape(B, L))


# speedup vs baseline: 1.0768x; 1.0768x over previous
import jax
import jax.numpy as jnp
from jax import lax
from jax.experimental import pallas as pl
from jax.experimental.pallas import tpu as pltpu
from jax.experimental.pallas import tpu_sc as plsc

SENT_LEN = 200
D = 64
NUM_WORKERS = 32
IDX_MINOR = 100
SENT_PER_CHUNK = 2
CHUNK = SENT_PER_CHUNK * SENT_LEN
ROWS_PER_CHUNK = CHUNK // IDX_MINOR
NBUF = 4
LANES = 16


def _sc_body(x2, xflat, table, pos2, postab, out, mask_out,
             pos_idx, pos_v, idx_g, idx_m, rows_v, mask_v,
             sem_pos, sem_g, sem_w):
    wid = lax.axis_index("s") * 2 + lax.axis_index("c")

    pltpu.sync_copy(pos2, pos_idx)
    for j in range(SENT_LEN // IDX_MINOR):
        pltpu.async_copy(postab.at[pos_idx.at[j]],
                         pos_v.at[pl.ds(j * IDX_MINOR, IDX_MINOR)],
                         sem_pos).wait()

    tokens_per_worker = x2.shape[0] * IDX_MINOR // NUM_WORKERS
    n_steps = tokens_per_worker // CHUNK
    tok0 = wid * tokens_per_worker

    def tok_base(s):
        return pl.multiple_of(tok0 + s * CHUNK, CHUNK)

    def row_base(s):
        return pl.multiple_of(tok_base(s) // IDX_MINOR, ROWS_PER_CHUNK)

    def load_and_fire(s, b):
        pltpu.sync_copy(x2.at[pl.ds(row_base(s), ROWS_PER_CHUNK)], idx_g[b])
        pltpu.sync_copy(xflat.at[pl.ds(tok_base(s), CHUNK)], idx_m[b])
        for j in range(ROWS_PER_CHUNK):
            pltpu.async_copy(table.at[idx_g[b].at[j]],
                             rows_v[b].at[pl.ds(j * IDX_MINOR, IDX_MINOR)],
                             sem_g[b])

    def wait_gathers(s, b):
        for j in range(ROWS_PER_CHUNK):
            pltpu.make_async_copy(
                table.at[idx_g[b].at[j]],
                rows_v[b].at[pl.ds(j * IDX_MINOR, IDX_MINOR)],
                sem_g[b]).wait()

    def wb_descriptor(s, b):
        return pltpu.make_async_copy(rows_v[b],
                                     out.at[pl.ds(tok_base(s), CHUNK)],
                                     sem_w[b])

    load_and_fire(0, 0)
    load_and_fire(1, 1)

    def iteration(i, carry):
        for b in range(NBUF):
            s = i * NBUF + b

            wait_gathers(s, b)

            @pl.loop(0, SENT_LEN)
            def add_body(r):
                for s2 in range(SENT_PER_CHUNK):
                    for col in range(D // LANES):
                        sl = pl.ds(col * LANES, LANES)
                        rows_v[b][s2 * SENT_LEN + r, sl] = (
                            rows_v[b][s2 * SENT_LEN + r, sl] + pos_v[r, sl])

            @pl.loop(0, CHUNK // LANES)
            def mask_body(j):
                sl = pl.ds(j * LANES, LANES)
                v = idx_m[b][sl]
                mask_v[b][sl] = jnp.where(v == jnp.int32(0), jnp.int32(1),
                                          jnp.int32(0))

            pltpu.sync_copy(mask_v[b], mask_out.at[pl.ds(tok_base(s), CHUNK)])
            wb_descriptor(s, b).start()

            bp = (b + 2) % NBUF

            if b >= 2:
                @pl.when(i < n_steps // NBUF - 1)
                def _prefetch():
                    wb_descriptor(s - 2, bp).wait()
                    load_and_fire(s + 2, bp)
            else:

                @pl.when(s + 2 < n_steps)
                def _prefetch():
                    @pl.when(s >= 2)
                    def _drain():
                        wb_descriptor(s - 2, bp).wait()
                    load_and_fire(s + 2, bp)
        return carry

    lax.fori_loop(0, n_steps // NBUF, iteration, 0)

    for b in range(NBUF):
        wb_descriptor(n_steps - NBUF + b, b).wait()


def kernel(x, token_table, pos_table, positions):
    B, L = x.shape
    n_tok = B * L
    x32 = x.astype(jnp.int32)
    x2 = x32.reshape(n_tok // IDX_MINOR, IDX_MINOR)
    xflat = x32.reshape(n_tok)
    pos2 = positions.astype(jnp.int32).reshape(L // IDX_MINOR, IDX_MINOR)

    mesh = plsc.VectorSubcoreMesh(core_axis_name="c", subcore_axis_name="s")
    out, mask_i32 = pl.kernel(
        _sc_body,
        out_type=[
            jax.ShapeDtypeStruct((n_tok, D), jnp.float32),
            jax.ShapeDtypeStruct((n_tok,), jnp.int32),
        ],
        mesh=mesh,
        compiler_params=pltpu.CompilerParams(use_tc_tiling_on_sc=False),
        scratch_types=[
            pltpu.VMEM((L // IDX_MINOR, IDX_MINOR), jnp.int32),
            pltpu.VMEM((SENT_LEN, D), jnp.float32),
            [pltpu.VMEM((ROWS_PER_CHUNK, IDX_MINOR), jnp.int32)
             for _ in range(NBUF)],
            [pltpu.VMEM((CHUNK,), jnp.int32) for _ in range(NBUF)],
            [pltpu.VMEM((CHUNK, D), jnp.float32) for _ in range(NBUF)],
            [pltpu.VMEM((CHUNK,), jnp.int32) for _ in range(NBUF)],
            pltpu.SemaphoreType.DMA,
            [pltpu.SemaphoreType.DMA for _ in range(NBUF)],
            [pltpu.SemaphoreType.DMA for _ in range(NBUF)],
        ],
    )(x2, xflat, token_table, pos2, pos_table)

    return (out.reshape(B, L, D), mask_i32.astype(jnp.bool_).reshape(B, L))
